# baseline (speedup 1.0000x reference)
import functools

import jax
import jax.numpy as jnp
from jax import lax
from jax.experimental import pallas as pl
from jax.experimental.pallas import tpu as pltpu
from jax.experimental.pallas import tpu_sc as plsc

N = 10000
E = 320000
H = 128
G = 64
C = 10

NC = 2
NS = 16
NW = NC * NS
EPW = E // NW
CH = 88
NFULL = EPW // CH
TAIL = EPW - NFULL * CH
RPT = 624
RREM = N - NS * RPT


NI = 8
NR = 4


def _sc_agg_body(h_hbm, src_hbm, dst_hbm, out_hbm, *s):
    srcs, dsts, rows = s[0:NI], s[NI:2 * NI], s[2 * NI:2 * NI + NR]
    si = s[2 * NI + NR:3 * NI + NR]
    sg = s[3 * NI + NR:3 * NI + 2 * NR]
    ss = s[3 * NI + 2 * NR:3 * NI + 3 * NR]
    src_t, dst_t, acc_sh = s[3 * NI + 3 * NR:]

    cid = lax.axis_index("c")
    sid = lax.axis_index("s")
    wid = sid * NC + cid

    base = wid * EPW

    def idx_start(j, ki):
        off = base + j * CH
        pltpu.async_copy(src_hbm.at[pl.ds(off, CH)], srcs[ki], si[ki])
        pltpu.async_copy(dst_hbm.at[pl.ds(off, CH)], dsts[ki], si[ki])

    def idx_wait(j, ki):
        off = base + j * CH
        pltpu.make_async_copy(src_hbm.at[pl.ds(off, CH)], srcs[ki],
                              si[ki]).wait()
        pltpu.make_async_copy(dst_hbm.at[pl.ds(off, CH)], dsts[ki],
                              si[ki]).wait()

    def gather_start(ki, kr):
        pltpu.async_copy(h_hbm.at[srcs[ki]], rows[kr], sg[kr])

    def gather_wait(ki, kr):
        pltpu.make_async_copy(h_hbm.at[srcs[ki]], rows[kr], sg[kr]).wait()

    def scatter_start(ki, kr):
        pltpu.async_copy(rows[kr], acc_sh.at[dsts[ki]], ss[kr], add=True)

    def scatter_wait(ki, kr):
        pltpu.make_async_copy(rows[kr], acc_sh.at[dsts[ki]], ss[kr]).wait()

    for jj in range(6):
        idx_start(jj, jj)
    for jj in range(2):
        idx_wait(jj, jj)
        gather_start(jj, jj)

    zv = jnp.zeros((16,), jnp.float32)

    def zrow(r, carry):
        for c in range(H // 16):
            rows[3][r, pl.ds(16 * c, 16)] = zv
        return carry

    lax.fori_loop(0, CH, zrow, 0)
    for m in range(RPT // CH):
        pltpu.sync_copy(rows[3], acc_sh.at[pl.ds(sid * RPT + m * CH, CH)])
    pltpu.sync_copy(rows[3].at[pl.ds(0, RPT % CH)],
                    acc_sh.at[pl.ds(sid * RPT + (RPT // CH) * CH, RPT % CH)])

    @pl.when(sid == 0)
    def _():
        pltpu.sync_copy(rows[3].at[pl.ds(0, RREM)],
                        acc_sh.at[pl.ds(NS * RPT, RREM)])

    plsc.subcore_barrier()

    def emit(j, k, when):
        when(j + 2 < NFULL, lambda: idx_wait(j + 2, (k + 2) % NI))
        when(j >= 2,
             lambda: scatter_wait((k + NI - 2) % NI, (k + 2) % NR))
        when(j + 2 < NFULL,
             lambda: gather_start((k + 2) % NI, (k + 2) % NR))
        when(j + 6 < NFULL, lambda: idx_start(j + 6, (k + 6) % NI))
        gather_wait(k % NI, k % NR)
        scatter_start(k % NI, k % NR)

    def traced_when(cond, fn):
        pl.when(cond)(fn)

    def static_when(cond, fn):
        if cond:
            fn()

    def group(g, carry):
        for k in range(NI):
            emit(g * NI + k, k, traced_when)
        return carry

    NG = NFULL // NI
    lax.fori_loop(0, NG, group, 0)
    for j in range(NG * NI, NFULL):
        emit(j, j % NI, static_when)
    scatter_wait((NFULL - 2) % NI, (NFULL - 2) % NR)
    scatter_wait((NFULL - 1) % NI, (NFULL - 1) % NR)

    toff = base + NFULL * CH
    pltpu.sync_copy(src_hbm.at[pl.ds(toff, TAIL)], src_t)
    pltpu.sync_copy(dst_hbm.at[pl.ds(toff, TAIL)], dst_t)
    pltpu.async_copy(h_hbm.at[src_t], rows[0].at[pl.ds(0, TAIL)], sg[0]).wait()
    pltpu.sync_copy(rows[0].at[pl.ds(0, TAIL)], acc_sh.at[dst_t], add=True)

    plsc.subcore_barrier()
    pltpu.sync_copy(acc_sh.at[pl.ds(sid * RPT, RPT)],
                    out_hbm.at[cid, pl.ds(sid * RPT, RPT)])

    @pl.when(sid == 0)
    def _():
        pltpu.sync_copy(acc_sh.at[pl.ds(NS * RPT, RREM)],
                        out_hbm.at[cid, pl.ds(NS * RPT, RREM)])


@functools.cache
def _get_sc_agg():
    return pl.kernel(
        _sc_agg_body,
        out_type=jax.ShapeDtypeStruct((NC, N, H), jnp.float32),
        mesh=plsc.VectorSubcoreMesh(core_axis_name="c", subcore_axis_name="s"),
        scratch_types=(
            [pltpu.VMEM((CH,), jnp.int32)] * NI
            + [pltpu.VMEM((CH,), jnp.int32)] * NI
            + [pltpu.VMEM((CH, H), jnp.float32)] * NR
            + [pltpu.SemaphoreType.DMA] * (NI + 2 * NR)
            + [
                pltpu.VMEM((TAIL,), jnp.int32),
                pltpu.VMEM((TAIL,), jnp.int32),
                pltpu.VMEM_SHARED((N, H), jnp.float32),
            ]
        ),
    )


BLK = 5000
NBLK = N // BLK
_INV_SQRT = 1.0 / (1.0 + 1e-5) ** 0.5


def _mlp_pool_body(h_ref, agg_ref, batch_ref, w1_ref, b1_ref, g_ref, be_ref,
                   w2_ref, b2_ref, hout_ref, p_ref):
    scale = g_ref[...] * _INV_SQRT
    w1s = w1_ref[...] * scale
    bias = b1_ref[...] * scale + be_ref[...]
    z = h_ref[...] + agg_ref[0] + agg_ref[1]
    a = jnp.dot(z, w1s, preferred_element_type=jnp.float32) + bias
    a = jnp.maximum(a, 0.0)
    o = jnp.dot(a, w2_ref[...], preferred_element_type=jnp.float32) + b2_ref[...]
    o = jnp.maximum(o, 0.0)
    hout_ref[...] = o

    ids = batch_ref[0, 0, :]
    oh = (lax.broadcasted_iota(jnp.int32, (G, BLK), 0)
          == ids[None, :]).astype(jnp.float32)
    ppart = jnp.dot(oh, o, preferred_element_type=jnp.float32)

    @pl.when(pl.program_id(0) == 0)
    def _():
        p_ref[...] = jnp.zeros_like(p_ref)

    p_ref[...] += ppart


_mlp_pool = pl.pallas_call(
    _mlp_pool_body,
    grid=(NBLK,),
    in_specs=[
        pl.BlockSpec((BLK, H), lambda i: (i, 0)),
        pl.BlockSpec((NC, BLK, H), lambda i: (0, i, 0)),
        pl.BlockSpec((1, 1, BLK), lambda i: (i, 0, 0)),
        pl.BlockSpec((H, H), lambda i: (0, 0)),
        pl.BlockSpec((1, H), lambda i: (0, 0)),
        pl.BlockSpec((1, H), lambda i: (0, 0)),
        pl.BlockSpec((1, H), lambda i: (0, 0)),
        pl.BlockSpec((H, H), lambda i: (0, 0)),
        pl.BlockSpec((1, H), lambda i: (0, 0)),
    ],
    out_specs=[
        pl.BlockSpec((BLK, H), lambda i: (i, 0)),
        pl.BlockSpec((G, H), lambda i: (0, 0)),
    ],
    out_shape=[
        jax.ShapeDtypeStruct((N, H), jnp.float32),
        jax.ShapeDtypeStruct((G, H), jnp.float32),
    ],
)


def _mlp_pool_head_body(h_ref, agg_ref, batch_ref, w1_ref, b1_ref, g_ref,
                        be_ref, w2_ref, b2_ref, p1_ref, p2_ref, l1w_ref,
                        l1b_ref, l2w_ref, l2b_ref, hout_ref, p_ref, out_ref):
    _mlp_pool_body(h_ref, agg_ref, batch_ref, w1_ref, b1_ref, g_ref, be_ref,
                   w2_ref, b2_ref, hout_ref, p_ref)

    @pl.when(pl.program_id(0) == NBLK - 1)
    def _():
        u = (jnp.dot(p1_ref[...], l1w_ref[0:H, :],
                     preferred_element_type=jnp.float32)
             + jnp.dot(p2_ref[...], l1w_ref[H:2 * H, :],
                       preferred_element_type=jnp.float32)
             + jnp.dot(p_ref[...], l1w_ref[2 * H:3 * H, :],
                       preferred_element_type=jnp.float32)
             + l1b_ref[...])
        u = jnp.maximum(u, 0.0)
        out_ref[...] = jnp.dot(u, l2w_ref[...],
                               preferred_element_type=jnp.float32) \
            + l2b_ref[...]


_FULL = pl.BlockSpec((G, H), lambda i: (0, 0))
_mlp_pool_head = pl.pallas_call(
    _mlp_pool_head_body,
    grid=(NBLK,),
    in_specs=[
        pl.BlockSpec((BLK, H), lambda i: (i, 0)),
        pl.BlockSpec((NC, BLK, H), lambda i: (0, i, 0)),
        pl.BlockSpec((1, 1, BLK), lambda i: (i, 0, 0)),
        pl.BlockSpec((H, H), lambda i: (0, 0)),
        pl.BlockSpec((1, H), lambda i: (0, 0)),
        pl.BlockSpec((1, H), lambda i: (0, 0)),
        pl.BlockSpec((1, H), lambda i: (0, 0)),
        pl.BlockSpec((H, H), lambda i: (0, 0)),
        pl.BlockSpec((1, H), lambda i: (0, 0)),
        _FULL,
        _FULL,
        pl.BlockSpec((3 * H, 3 * H), lambda i: (0, 0)),
        pl.BlockSpec((1, 3 * H), lambda i: (0, 0)),
        pl.BlockSpec((3 * H, C), lambda i: (0, 0)),
        pl.BlockSpec((1, C), lambda i: (0, 0)),
    ],
    out_specs=[
        pl.BlockSpec((BLK, H), lambda i: (i, 0)),
        pl.BlockSpec((G, H), lambda i: (0, 0)),
        pl.BlockSpec((G, C), lambda i: (0, 0)),
    ],
    out_shape=[
        jax.ShapeDtypeStruct((N, H), jnp.float32),
        jax.ShapeDtypeStruct((G, H), jnp.float32),
        jax.ShapeDtypeStruct((G, C), jnp.float32),
    ],
)


def kernel(x, edge_index, batch, c1_w1, c1_b1, c1_g, c1_be, c1_w2, c1_b2,
           c2_w1, c2_b1, c2_g, c2_be, c2_w2, c2_b2,
           c3_w1, c3_b1, c3_g, c3_be, c3_w2, c3_b2,
           l1_w, l1_b, l2_w, l2_b):
    src = edge_index[0].astype(jnp.int32)
    dst = edge_index[1].astype(jnp.int32)
    batch3 = batch.astype(jnp.int32).reshape(NBLK, 1, BLK)

    def row(v):
        return v.reshape(1, -1)

    def conv(h, w1, b1, g, be, w2, b2):
        aggp = _get_sc_agg()(h, src, dst)
        return _mlp_pool(h, aggp, batch3, w1, row(b1), row(g), row(be),
                         w2, row(b2))

    h1, p1 = conv(x, c1_w1, c1_b1, c1_g, c1_be, c1_w2, c1_b2)
    h2, p2 = conv(h1, c2_w1, c2_b1, c2_g, c2_be, c2_w2, c2_b2)
    agg3 = _get_sc_agg()(h2, src, dst)
    _, _, out = _mlp_pool_head(h2, agg3, batch3, c3_w1, row(c3_b1),
                               row(c3_g), row(c3_be), c3_w2, row(c3_b2),
                               p1, p2, l1_w, row(l1_b), l2_w, row(l2_b))
    return out


# speedup vs baseline: 1.0057x; 1.0057x over previous
import functools

import jax
import jax.numpy as jnp
from jax import lax
from jax.experimental import pallas as pl
from jax.experimental.pallas import tpu as pltpu
from jax.experimental.pallas import tpu_sc as plsc

N = 10000
E = 320000
H = 128
G = 64
C = 10

NC = 2
NS = 16
NW = NC * NS
EPW = E // NW
CH = 80
NFULL = EPW // CH
RPT = 624
RREM = N - NS * RPT


NI = 8
NR = 4


def _sc_agg_body(h_hbm, src_hbm, dst_hbm, out_hbm, *s):
    srcs, dsts, rows = s[0:NI], s[NI:2 * NI], s[2 * NI:2 * NI + NR]
    si = s[2 * NI + NR:3 * NI + NR]
    sg = s[3 * NI + NR:3 * NI + 2 * NR]
    ss = s[3 * NI + 2 * NR:3 * NI + 3 * NR]
    acc_sh = s[3 * NI + 3 * NR]

    cid = lax.axis_index("c")
    sid = lax.axis_index("s")
    wid = sid * NC + cid

    base = wid * EPW

    def idx_start(j, ki):
        off = base + j * CH
        pltpu.async_copy(src_hbm.at[pl.ds(off, CH)], srcs[ki], si[ki])
        pltpu.async_copy(dst_hbm.at[pl.ds(off, CH)], dsts[ki], si[ki])

    def idx_wait(j, ki):
        off = base + j * CH
        pltpu.make_async_copy(src_hbm.at[pl.ds(off, CH)], srcs[ki],
                              si[ki]).wait()
        pltpu.make_async_copy(dst_hbm.at[pl.ds(off, CH)], dsts[ki],
                              si[ki]).wait()

    def gather_start(ki, kr):
        pltpu.async_copy(h_hbm.at[srcs[ki]], rows[kr], sg[kr])

    def gather_wait(ki, kr):
        pltpu.make_async_copy(h_hbm.at[srcs[ki]], rows[kr], sg[kr]).wait()

    def scatter_start(ki, kr):
        pltpu.async_copy(rows[kr], acc_sh.at[dsts[ki]], ss[kr], add=True)

    def scatter_wait(ki, kr):
        pltpu.make_async_copy(rows[kr], acc_sh.at[dsts[ki]], ss[kr]).wait()

    for jj in range(6):
        idx_start(jj, jj)
    for jj in range(2):
        idx_wait(jj, jj)
        gather_start(jj, jj)

    zv = jnp.zeros((16,), jnp.float32)

    def zrow(r, carry):
        for c in range(H // 16):
            rows[3][r, pl.ds(16 * c, 16)] = zv
        return carry

    lax.fori_loop(0, CH, zrow, 0)
    for m in range(RPT // CH):
        pltpu.sync_copy(rows[3], acc_sh.at[pl.ds(sid * RPT + m * CH, CH)])
    pltpu.sync_copy(rows[3].at[pl.ds(0, RPT % CH)],
                    acc_sh.at[pl.ds(sid * RPT + (RPT // CH) * CH, RPT % CH)])

    @pl.when(sid == 0)
    def _():
        pltpu.sync_copy(rows[3].at[pl.ds(0, RREM)],
                        acc_sh.at[pl.ds(NS * RPT, RREM)])

    plsc.subcore_barrier()

    def emit(j, k, when):
        when(j + 2 < NFULL, lambda: idx_wait(j + 2, (k + 2) % NI))
        when(j >= 2,
             lambda: scatter_wait((k + NI - 2) % NI, (k + 2) % NR))
        when(j + 2 < NFULL,
             lambda: gather_start((k + 2) % NI, (k + 2) % NR))
        when(j + 6 < NFULL, lambda: idx_start(j + 6, (k + 6) % NI))
        gather_wait(k % NI, k % NR)
        scatter_start(k % NI, k % NR)

    def traced_when(cond, fn):
        pl.when(cond)(fn)

    def static_when(cond, fn):
        if cond:
            fn()

    def group(g, carry):
        for k in range(NI):
            emit(g * NI + k, k, traced_when)
        return carry

    NG = NFULL // NI
    lax.fori_loop(0, NG, group, 0)
    for j in range(NG * NI, NFULL):
        emit(j, j % NI, static_when)
    scatter_wait((NFULL - 2) % NI, (NFULL - 2) % NR)
    scatter_wait((NFULL - 1) % NI, (NFULL - 1) % NR)

    plsc.subcore_barrier()
    pltpu.sync_copy(acc_sh.at[pl.ds(sid * RPT, RPT)],
                    out_hbm.at[cid, pl.ds(sid * RPT, RPT)])

    @pl.when(sid == 0)
    def _():
        pltpu.sync_copy(acc_sh.at[pl.ds(NS * RPT, RREM)],
                        out_hbm.at[cid, pl.ds(NS * RPT, RREM)])


@functools.cache
def _get_sc_agg():
    return pl.kernel(
        _sc_agg_body,
        out_type=jax.ShapeDtypeStruct((NC, N, H), jnp.float32),
        mesh=plsc.VectorSubcoreMesh(core_axis_name="c", subcore_axis_name="s"),
        scratch_types=(
            [pltpu.VMEM((CH,), jnp.int32)] * NI
            + [pltpu.VMEM((CH,), jnp.int32)] * NI
            + [pltpu.VMEM((CH, H), jnp.float32)] * NR
            + [pltpu.SemaphoreType.DMA] * (NI + 2 * NR)
            + [pltpu.VMEM_SHARED((N, H), jnp.float32)]
        ),
    )


BLK = 5000
NBLK = N // BLK
_INV_SQRT = 1.0 / (1.0 + 1e-5) ** 0.5


def _mlp_pool_body(h_ref, agg_ref, batch_ref, w1_ref, b1_ref, g_ref, be_ref,
                   w2_ref, b2_ref, hout_ref, p_ref):
    scale = g_ref[...] * _INV_SQRT
    w1s = w1_ref[...] * scale
    bias = b1_ref[...] * scale + be_ref[...]
    z = h_ref[...] + agg_ref[0] + agg_ref[1]
    a = jnp.dot(z, w1s, preferred_element_type=jnp.float32) + bias
    a = jnp.maximum(a, 0.0)
    o = jnp.dot(a, w2_ref[...], preferred_element_type=jnp.float32) + b2_ref[...]
    o = jnp.maximum(o, 0.0)
    hout_ref[...] = o

    ids = batch_ref[0, 0, :]
    oh = (lax.broadcasted_iota(jnp.int32, (G, BLK), 0)
          == ids[None, :]).astype(jnp.float32)
    ppart = jnp.dot(oh, o, preferred_element_type=jnp.float32)

    @pl.when(pl.program_id(0) == 0)
    def _():
        p_ref[...] = jnp.zeros_like(p_ref)

    p_ref[...] += ppart


_mlp_pool = pl.pallas_call(
    _mlp_pool_body,
    grid=(NBLK,),
    in_specs=[
        pl.BlockSpec((BLK, H), lambda i: (i, 0)),
        pl.BlockSpec((NC, BLK, H), lambda i: (0, i, 0)),
        pl.BlockSpec((1, 1, BLK), lambda i: (i, 0, 0)),
        pl.BlockSpec((H, H), lambda i: (0, 0)),
        pl.BlockSpec((1, H), lambda i: (0, 0)),
        pl.BlockSpec((1, H), lambda i: (0, 0)),
        pl.BlockSpec((1, H), lambda i: (0, 0)),
        pl.BlockSpec((H, H), lambda i: (0, 0)),
        pl.BlockSpec((1, H), lambda i: (0, 0)),
    ],
    out_specs=[
        pl.BlockSpec((BLK, H), lambda i: (i, 0)),
        pl.BlockSpec((G, H), lambda i: (0, 0)),
    ],
    out_shape=[
        jax.ShapeDtypeStruct((N, H), jnp.float32),
        jax.ShapeDtypeStruct((G, H), jnp.float32),
    ],
)


def _mlp_pool_head_body(h_ref, agg_ref, batch_ref, w1_ref, b1_ref, g_ref,
                        be_ref, w2_ref, b2_ref, p1_ref, p2_ref, l1w_ref,
                        l1b_ref, l2w_ref, l2b_ref, hout_ref, p_ref, out_ref):
    _mlp_pool_body(h_ref, agg_ref, batch_ref, w1_ref, b1_ref, g_ref, be_ref,
                   w2_ref, b2_ref, hout_ref, p_ref)

    @pl.when(pl.program_id(0) == NBLK - 1)
    def _():
        u = (jnp.dot(p1_ref[...], l1w_ref[0:H, :],
                     preferred_element_type=jnp.float32)
             + jnp.dot(p2_ref[...], l1w_ref[H:2 * H, :],
                       preferred_element_type=jnp.float32)
             + jnp.dot(p_ref[...], l1w_ref[2 * H:3 * H, :],
                       preferred_element_type=jnp.float32)
             + l1b_ref[...])
        u = jnp.maximum(u, 0.0)
        out_ref[...] = jnp.dot(u, l2w_ref[...],
                               preferred_element_type=jnp.float32) \
            + l2b_ref[...]


_FULL = pl.BlockSpec((G, H), lambda i: (0, 0))
_mlp_pool_head = pl.pallas_call(
    _mlp_pool_head_body,
    grid=(NBLK,),
    in_specs=[
        pl.BlockSpec((BLK, H), lambda i: (i, 0)),
        pl.BlockSpec((NC, BLK, H), lambda i: (0, i, 0)),
        pl.BlockSpec((1, 1, BLK), lambda i: (i, 0, 0)),
        pl.BlockSpec((H, H), lambda i: (0, 0)),
        pl.BlockSpec((1, H), lambda i: (0, 0)),
        pl.BlockSpec((1, H), lambda i: (0, 0)),
        pl.BlockSpec((1, H), lambda i: (0, 0)),
        pl.BlockSpec((H, H), lambda i: (0, 0)),
        pl.BlockSpec((1, H), lambda i: (0, 0)),
        _FULL,
        _FULL,
        pl.BlockSpec((3 * H, 3 * H), lambda i: (0, 0)),
        pl.BlockSpec((1, 3 * H), lambda i: (0, 0)),
        pl.BlockSpec((3 * H, C), lambda i: (0, 0)),
        pl.BlockSpec((1, C), lambda i: (0, 0)),
    ],
    out_specs=[
        pl.BlockSpec((BLK, H), lambda i: (i, 0)),
        pl.BlockSpec((G, H), lambda i: (0, 0)),
        pl.BlockSpec((G, C), lambda i: (0, 0)),
    ],
    out_shape=[
        jax.ShapeDtypeStruct((N, H), jnp.float32),
        jax.ShapeDtypeStruct((G, H), jnp.float32),
        jax.ShapeDtypeStruct((G, C), jnp.float32),
    ],
)


def kernel(x, edge_index, batch, c1_w1, c1_b1, c1_g, c1_be, c1_w2, c1_b2,
           c2_w1, c2_b1, c2_g, c2_be, c2_w2, c2_b2,
           c3_w1, c3_b1, c3_g, c3_be, c3_w2, c3_b2,
           l1_w, l1_b, l2_w, l2_b):
    src = edge_index[0].astype(jnp.int32)
    dst = edge_index[1].astype(jnp.int32)
    batch3 = batch.astype(jnp.int32).reshape(NBLK, 1, BLK)

    def row(v):
        return v.reshape(1, -1)

    def conv(h, w1, b1, g, be, w2, b2):
        aggp = _get_sc_agg()(h, src, dst)
        return _mlp_pool(h, aggp, batch3, w1, row(b1), row(g), row(be),
                         w2, row(b2))

    h1, p1 = conv(x, c1_w1, c1_b1, c1_g, c1_be, c1_w2, c1_b2)
    h2, p2 = conv(h1, c2_w1, c2_b1, c2_g, c2_be, c2_w2, c2_b2)
    agg3 = _get_sc_agg()(h2, src, dst)
    _, _, out = _mlp_pool_head(h2, agg3, batch3, c3_w1, row(c3_b1),
                               row(c3_g), row(c3_be), c3_w2, row(c3_b2),
                               p1, p2, l1_w, row(l1_b), l2_w, row(l2_b))
    return out
